# baseline (speedup 1.0000x reference)
import jax
import jax.numpy as jnp
from jax.experimental import pallas as pl
from jax.experimental.pallas import tpu as pltpu

_NUM_ENH = 8
_BB = 8


def _body(feat_ref, out_ref, tail_ref, copy_sem, tail_sem):
    i = pl.program_id(0)
    s, d = feat_ref.shape[1], feat_ref.shape[2]
    copy = pltpu.make_async_copy(
        feat_ref,
        out_ref.at[pl.ds(i * _BB, _BB), pl.ds(0, s), :],
        copy_sem,
    )
    copy.start()
    for b in range(_BB):
        total = jnp.sum(feat_ref[b], axis=0, keepdims=True)
        tail_ref[b] = jnp.broadcast_to(total, (_NUM_ENH, d))
    tail = pltpu.make_async_copy(
        tail_ref,
        out_ref.at[pl.ds(i * _BB, _BB), pl.ds(s, _NUM_ENH), :],
        tail_sem,
    )
    tail.start()
    copy.wait()
    tail.wait()


def kernel(features, alpha):
    del alpha
    B, S, D = features.shape
    return pl.pallas_call(
        _body,
        grid=(B // _BB,),
        in_specs=[pl.BlockSpec((_BB, S, D), lambda i: (i, 0, 0))],
        out_specs=pl.BlockSpec(memory_space=pltpu.MemorySpace.HBM),
        out_shape=jax.ShapeDtypeStruct((B, S + _NUM_ENH, D), features.dtype),
        scratch_shapes=[
            pltpu.MemorySpace.VMEM((_BB, _NUM_ENH, D), jnp.float32),
            pltpu.SemaphoreType.DMA,
            pltpu.SemaphoreType.DMA,
        ],
        compiler_params=pltpu.CompilerParams(
            dimension_semantics=("arbitrary",),
            vmem_limit_bytes=100 * 1024 * 1024,
        ),
    )(features)
